# baseline (speedup 1.0000x reference)
import functools

import jax
import jax.numpy as jnp
from jax import lax
from jax.experimental import pallas as pl
from jax.experimental.pallas import tpu as pltpu
from jax.experimental.pallas import tpu_sc as plsc

VOCAB = 1_000_000
EMBED = 64
BATCH = 4096
SEQ = 200

NC, NS = 2, 16
NW = NC * NS
ROWS_PER_W = BATCH // NW

MV_ROWS = VOCAB // 2
MV_BLK = 4000


def _matvec_body(t_ref, w_ref, b_ref, o_ref):
    o_ref[...] = (
        jnp.dot(t_ref[...], w_ref[...], preferred_element_type=jnp.float32)
        + b_ref[0, 0]
    )


def _tw_matvec(table2, wtile, b2):
    return pl.pallas_call(
        _matvec_body,
        grid=(MV_ROWS // MV_BLK,),
        in_specs=[
            pl.BlockSpec((MV_BLK, 128), lambda i: (i, 0)),
            pl.BlockSpec((128, 2), lambda i: (0, 0)),
            pl.BlockSpec((1, 1), lambda i: (0, 0)),
        ],
        out_specs=pl.BlockSpec((MV_BLK, 2), lambda i: (i, 0)),
        out_shape=jax.ShapeDtypeStruct((MV_ROWS, 2), jnp.float32),
    )(table2, wtile, b2)


_SC_MESH = plsc.VectorSubcoreMesh(core_axis_name="c", subcore_axis_name="s")


@functools.partial(
    pl.kernel,
    out_type=jax.ShapeDtypeStruct((BATCH,), jnp.float32),
    mesh=_SC_MESH,
    scratch_types=[
        pltpu.VMEM((SEQ * ROWS_PER_W,), jnp.int32),
        pltpu.VMEM((SEQ * ROWS_PER_W,), jnp.float32),
        pltpu.VMEM((ROWS_PER_W,), jnp.float32),
        pltpu.SemaphoreType.DMA,
    ],
)
def _sc_pool(xr_hbm, tw_hbm, out_hbm, idx_v, vals_v, res_v, sem):
    wid = lax.axis_index("s") * NC + lax.axis_index("c")
    pltpu.sync_copy(xr_hbm.at[wid], idx_v)
    pltpu.async_copy(tw_hbm.at[idx_v], vals_v, sem).wait()

    nsub = ROWS_PER_W // 16

    def body(j, accs):
        base = j * ROWS_PER_W
        return tuple(
            accs[k] + vals_v[pl.ds(base + k * 16, 16)] for k in range(nsub)
        )

    accs = lax.fori_loop(
        0, SEQ, body, tuple(jnp.zeros((16,), jnp.float32) for _ in range(nsub))
    )
    inv = jnp.float32(1.0 / SEQ)
    for k in range(nsub):
        z = accs[k] * inv
        res_v[pl.ds(k * 16, 16)] = 1.0 / (1.0 + jnp.exp(-z))
    pltpu.sync_copy(res_v, out_hbm.at[pl.ds(wid * ROWS_PER_W, ROWS_PER_W)])


def kernel(x, table, W, b):
    table2 = table.reshape(MV_ROWS, 128)
    w = W[:, 0]
    wtile = (
        jnp.zeros((128, 2), jnp.float32)
        .at[0:64, 0].set(w)
        .at[64:128, 1].set(w)
    )
    b2 = b.reshape(1, 1)
    tw = _tw_matvec(table2, wtile, b2).reshape(VOCAB)

    xr = jnp.transpose(
        x.astype(jnp.int32).reshape(NW, ROWS_PER_W, SEQ), (0, 2, 1)
    ).reshape(NW, SEQ * ROWS_PER_W)
    out = _sc_pool(xr, tw)
    return out.reshape(BATCH, 1)
